# Initial kernel scaffold; baseline (speedup 1.0000x reference)
import jax
import jax.numpy as jnp
from jax.experimental import pallas as pl


def kernel(x, params):
    raise NotImplementedError("write your pallas kernel here")


# speedup vs baseline: 3.3072x; 3.3072x over previous
import jax
import jax.numpy as jnp
from jax.experimental import pallas as pl
from jax.experimental.pallas import tpu as pltpu

N = 128
K = 30
NLAYERS = 4
DSTATE = 12
DTRANK = 8
DCONV = 4
B = 2
FDIM = 480
T = 640
BW = FDIM // K
S = B * K
TC2 = 128
F32 = jnp.float32


def _ln(x, w, b):
    mu = jnp.mean(x, axis=-1, keepdims=True)
    v = jnp.mean(x * x, axis=-1, keepdims=True) - mu * mu
    return (x - mu) * jax.lax.rsqrt(v + 1e-5) * w + b


def _sigmoid(x):
    return 1.0 / (1.0 + jnp.exp(-x))


def _silu(x):
    return x * _sigmoid(x)


def _softplus(x):
    return jnp.maximum(x, 0.0) + jnp.log(1.0 + jnp.exp(-jnp.abs(x)))


def _erf(x):
    a1, a2, a3 = 0.254829592, -0.284496736, 1.421413741
    a4, a5, p = -1.453152027, 1.061405429, 0.3275911
    sgn = jnp.where(x < 0, -1.0, 1.0)
    ax = jnp.abs(x)
    t = 1.0 / (1.0 + p * ax)
    poly = ((((a5 * t + a4) * t + a3) * t + a2) * t + a1) * t
    return sgn * (1.0 - poly * jnp.exp(-ax * ax))


def _gelu(x):
    return 0.5 * x * (1.0 + _erf(x * 0.7071067811865476))


def _dotT(x, w):
    return jax.lax.dot_general(x, w, (((1,), (1,)), ((), ())),
                               preferred_element_type=F32)


def _encoder_kernel(feat_ref, fcw_ref, fcb_ref, blnw_ref, blnb_ref,
                    mlnw_ref, mlnb_ref, inw_ref, cwt_ref, cb_ref,
                    xpw_ref, dtw_ref, dtb_ref, alog_ref, dmd_ref,
                    outw_ref, s2_ref,
                    z_ref, out_ref,
                    dA_s, dBu_s, H_s):
    f = feat_ref[0]
    fn = _ln(f, blnw_ref[0], blnb_ref[0])
    z = _dotT(fn, fcw_ref[0]) + fcb_ref[0]
    z_ref[0] = z

    xs = z
    for i in range(NLAYERS):
        xn = _ln(xs, mlnw_ref[i], mlnb_ref[i])
        xz = _dotT(xn, inw_ref[i])
        xp0 = xz[:, :N]
        zg = xz[:, N:]
        xc = cwt_ref[i, 3][None, :] * xp0
        for j in (1, 2, 3):
            shifted = jnp.concatenate(
                [jnp.zeros((j, N), F32), xp0[:T - j]], axis=0)
            xc = xc + cwt_ref[i, 3 - j][None, :] * shifted
        xp = _silu(xc + cb_ref[i])
        dbl = _dotT(xp, xpw_ref[i])
        dt = _softplus(_dotT(dbl[:, :DTRANK], dtw_ref[i]) + dtb_ref[i])
        Am = -jnp.exp(alog_ref[i])
        dA_s[...] = jnp.exp(dt[:, None, :] * Am[None])
        s2 = s2_ref[...]
        bmf = jax.lax.dot_general(
            dbl[:, DTRANK:DTRANK + DSTATE], s2, (((1,), (0,)), ((), ())),
            preferred_element_type=F32)
        cmf = jax.lax.dot_general(
            dbl[:, DTRANK + DSTATE:], s2, (((1,), (0,)), ((), ())),
            preferred_element_type=F32)
        du = dt * xp
        bm3 = jnp.concatenate(
            [bmf[:, n * N:(n + 1) * N][:, None, :] for n in range(DSTATE)],
            axis=1)
        dBu_s[...] = du[:, None, :] * bm3

        def body(t, h):
            h = dA_s[t] * h + dBu_s[t]
            H_s[t] = h
            return h

        jax.lax.fori_loop(0, T, body, jnp.zeros((DSTATE, N), F32))

        Hv = H_s[...]
        y = xp * dmd_ref[i]
        for n in range(DSTATE):
            y = y + Hv[:, n, :] * cmf[:, n * N:(n + 1) * N]
        ys = y * _silu(zg)
        xs = xs + _dotT(ys, outw_ref[i])
    out_ref[0] = xs


def _head_kernel(oc_ref, z_ref, cbw_ref, cbb_ref, cblnw_ref, cblnb_ref,
                 cbnw_ref, cbnb_ref, magw_ref, magb_ref,
                 w1_ref, b1_ref, w2_ref, b2_ref, bmw_ref, bias_ref,
                 out_ref):
    oc = oc_ref[0].reshape(K * TC2, N)
    zt = z_ref[0].reshape(K * TC2, N)
    oc2 = _gelu(_ln(_dotT(oc, cbw_ref[...]) + cbb_ref[...],
                    cblnw_ref[...], cblnb_ref[...]))
    feats = _ln(oc + oc2, cbnw_ref[...], cbnb_ref[...])
    mag = _sigmoid(_dotT(feats, magw_ref[...]) + magb_ref[...])
    zm = zt * mag
    hh = jnp.tanh(_dotT(zm, w1_ref[...]) + b1_ref[...])
    hh = _dotT(hh, w2_ref[...]) + b2_ref[...]
    h3 = hh.reshape(K, TC2, N)
    acc = bias_ref[...]
    for k in range(K):
        acc = acc + jax.lax.dot_general(
            bmw_ref[k], h3[k], (((1,), (1,)), ((), ())),
            preferred_element_type=F32)
    out_ref[0] = acc


def kernel(x, params):
    p = params
    xr = x[:, 0].reshape(B, K, BW, T)
    xi = x[:, 1].reshape(B, K, BW, T)
    feat = jnp.concatenate([xr, xi], axis=2)
    feat = feat.transpose(0, 1, 3, 2).reshape(S, T, 2 * BW)

    blnw = p['bs_ln_w'][:, None, :]
    blnb = p['bs_ln_b'][:, None, :]
    fcb = p['bs_fc_b'][:, None, :]
    mlnw = p['m_ln_w'][:, None, :]
    mlnb = p['m_ln_b'][:, None, :]
    cwt = p['m_conv_w'].transpose(0, 2, 1)
    cbias = p['m_conv_b'][:, None, :]
    dtb = p['m_dt_b'][:, None, :]
    alogT = p['m_A_log'].transpose(0, 2, 1)
    dmd = p['m_D'][:, None, :]
    s2 = jnp.repeat(jnp.eye(DSTATE, dtype=F32), N, axis=1)

    full = lambda shape: pl.BlockSpec(shape, lambda s_: tuple(0 for _ in shape))

    z_seq, out_seq = pl.pallas_call(
        _encoder_kernel,
        grid=(S,),
        in_specs=[
            pl.BlockSpec((1, T, 2 * BW), lambda s_: (s_, 0, 0)),
            pl.BlockSpec((1, N, 2 * BW), lambda s_: (jax.lax.rem(s_, K), 0, 0)),
            pl.BlockSpec((1, 1, N), lambda s_: (jax.lax.rem(s_, K), 0, 0)),
            pl.BlockSpec((1, 1, 2 * BW), lambda s_: (jax.lax.rem(s_, K), 0, 0)),
            pl.BlockSpec((1, 1, 2 * BW), lambda s_: (jax.lax.rem(s_, K), 0, 0)),
            full((NLAYERS, 1, N)),
            full((NLAYERS, 1, N)),
            full((NLAYERS, 2 * N, N)),
            full((NLAYERS, DCONV, N)),
            full((NLAYERS, 1, N)),
            full((NLAYERS, DTRANK + 2 * DSTATE, N)),
            full((NLAYERS, N, DTRANK)),
            full((NLAYERS, 1, N)),
            full((NLAYERS, DSTATE, N)),
            full((NLAYERS, 1, N)),
            full((NLAYERS, N, N)),
            full((DSTATE, DSTATE * N)),
        ],
        out_specs=[
            pl.BlockSpec((1, T, N), lambda s_: (s_, 0, 0)),
            pl.BlockSpec((1, T, N), lambda s_: (s_, 0, 0)),
        ],
        out_shape=[
            jax.ShapeDtypeStruct((S, T, N), F32),
            jax.ShapeDtypeStruct((S, T, N), F32),
        ],
        scratch_shapes=[
            pltpu.VMEM((T, DSTATE, N), F32),
            pltpu.VMEM((T, DSTATE, N), F32),
            pltpu.VMEM((T, DSTATE, N), F32),
        ],
        compiler_params=pltpu.CompilerParams(
            dimension_semantics=("parallel",),
            vmem_limit_bytes=56 * 1024 * 1024,
        ),
    )(feat, p['bs_fc_w'], fcb, blnw, blnb,
      mlnw, mlnb, p['m_in_w'], cwt, cbias,
      p['m_xproj_w'], p['m_dt_w'], dtb, alogT, dmd,
      p['m_out_w'], s2)

    oc4 = out_seq.reshape(B, K, T, N)
    z4 = z_seq.reshape(B, K, T, N)
    cbb = p['cb_b'][None, :]
    cblnw = p['cb_ln_w'][None, :]
    cblnb = p['cb_ln_b'][None, :]
    cbnw = p['cbn_w'][None, :]
    cbnb = p['cbn_b'][None, :]
    magb = p['mag_b'][None, :]
    b1 = p['dec_b1'][None, :]
    b2 = p['dec_b2'][None, :]
    bias = jnp.broadcast_to(jnp.sum(p['bm_b'], axis=0)[:, None], (N, TC2))

    full2 = lambda shape: pl.BlockSpec(shape, lambda b_, t_: tuple(0 for _ in shape))

    enhanced = pl.pallas_call(
        _head_kernel,
        grid=(B, T // TC2),
        in_specs=[
            pl.BlockSpec((1, K, TC2, N), lambda b_, t_: (b_, 0, t_, 0)),
            pl.BlockSpec((1, K, TC2, N), lambda b_, t_: (b_, 0, t_, 0)),
            full2((N, N)), full2((1, N)), full2((1, N)), full2((1, N)),
            full2((1, N)), full2((1, N)), full2((N, N)), full2((1, N)),
            full2((2 * N, N)), full2((1, 2 * N)),
            full2((N, 2 * N)), full2((1, N)),
            full2((K, N, N)), full2((N, TC2)),
        ],
        out_specs=pl.BlockSpec((1, N, TC2), lambda b_, t_: (b_, 0, t_)),
        out_shape=jax.ShapeDtypeStruct((B, N, T), F32),
        compiler_params=pltpu.CompilerParams(
            dimension_semantics=("parallel", "parallel"),
        ),
    )(oc4, z4, p['cb_w'], cbb, cblnw, cblnb, cbnw, cbnb,
      p['mag_w'], magb, p['dec_w1'], b1, p['dec_w2'], b2,
      p['bm_w'], bias)

    return enhanced
